# baseline (speedup 1.0000x reference)
import math

import numpy as np
import jax
import jax.numpy as jnp
from jax.experimental import pallas as pl
from jax.experimental.pallas import tpu as pltpu

_B, _D, _K, _H = 512, 256, 1024, 32
_TB = 128
_NB = _B // _TB
_KB = 128


def _make_gumbel():
    n = _B * _K
    ks0, ks1 = np.uint32(0), np.uint32(42)
    ks2 = ks0 ^ ks1 ^ np.uint32(0x1BD11BDA)
    x0 = np.full(n, ks0, np.uint32)
    x1 = np.arange(n, dtype=np.uint32)
    rot_a = np.uint32([13, 15, 26, 6])
    rot_b = np.uint32([17, 29, 16, 24])

    def rotl(v, d):
        return (v << d) | (v >> np.uint32(32 - d))

    with np.errstate(over="ignore"):
        x1 = x1 + ks1
        for rs, ka, kb, c in ((rot_a, ks1, ks2, 1), (rot_b, ks2, ks0, 2),
                              (rot_a, ks0, ks1, 3), (rot_b, ks1, ks2, 4),
                              (rot_a, ks2, ks0, 5)):
            for r in rs:
                x0 = x0 + x1
                x1 = x0 ^ rotl(x1, r)
            x0 = x0 + ka
            x1 = x1 + kb + np.uint32(c)
    bits = x0 ^ x1
    fb = (bits >> np.uint32(9)) | np.uint32(0x3F800000)
    tiny = np.float32(np.finfo(np.float32).tiny)
    u = np.maximum(tiny, (fb.view(np.float32) - np.float32(1.0))
                   * (np.float32(1.0) - tiny) + tiny)
    return (-np.log(-np.log(u))).astype(np.float32).reshape(_B, _K)


_G = _make_gumbel()


def _vq_body(tau_ref, x_ref, cb_ref, cbt_ref, w1_ref, b1_ref, w2_ref, b2_ref,
             g_ref, logits_ref, bmu_ref, dpart_ref):
    x = x_ref[...]

    h = jnp.maximum(
        jax.lax.dot_general(x, w1_ref[...], (((1,), (0,)), ((), ())),
                            preferred_element_type=jnp.float32,
                            precision=jax.lax.Precision.HIGHEST)
        + b1_ref[...], 0.0)
    logits = jax.lax.dot_general(h, w2_ref[...], (((1,), (0,)), ((), ())),
                                 preferred_element_type=jnp.float32,
                                 precision=jax.lax.Precision.HIGHEST)
    logits = logits + b2_ref[...]
    logits_ref[...] = logits

    s = (logits + g_ref[...]) / tau_ref[0, 0]
    m = jnp.max(s, axis=1, keepdims=True)
    e = jnp.exp(s - m)
    z = e / jnp.sum(e, axis=1, keepdims=True)

    cbt = cbt_ref[...]
    cnorm = jnp.sum(cbt * cbt, axis=0, keepdims=True)
    dist = cnorm - 2.0 * jax.lax.dot_general(
        x, cbt, (((1,), (0,)), ((), ())),
        preferred_element_type=jnp.float32,
        precision=jax.lax.Precision.HIGHEST)
    iot = jax.lax.broadcasted_iota(jnp.int32, (_TB, _K), 1)
    m1 = jnp.min(dist, axis=1, keepdims=True)
    i1 = jnp.min(jnp.where(dist <= m1, iot, _K), axis=1, keepdims=True)
    dist2 = jnp.where(iot == i1, jnp.inf, dist)
    m2 = jnp.min(dist2, axis=1, keepdims=True)
    i2 = jnp.min(jnp.where(dist2 <= m2, iot, _K), axis=1, keepdims=True)
    ia = jnp.minimum(i1, i2)
    ib = jnp.maximum(i1, i2)
    cb = cb_ref[...]
    oha = (iot == ia).astype(jnp.float32)
    ohb = (iot == ib).astype(jnp.float32)
    ca = jax.lax.dot_general(oha, cb, (((1,), (0,)), ((), ())),
                             preferred_element_type=jnp.float32,
                             precision=jax.lax.Precision.HIGHEST)
    cbs = jax.lax.dot_general(ohb, cb, (((1,), (0,)), ((), ())),
                              preferred_element_type=jnp.float32,
                              precision=jax.lax.Precision.HIGHEST)
    da = ca - x
    db = cbs - x
    ssqa = jnp.sum(da * da, axis=1, keepdims=True)
    ssqb = jnp.sum(db * db, axis=1, keepdims=True)
    bmu_ref[...] = jnp.where(ssqa <= ssqb, ia, ib)[None]

    cbt_bf = cbt.astype(jnp.bfloat16)
    xbc = x.astype(jnp.bfloat16)[:, :, None]
    accm = jnp.zeros((_TB, _KB), jnp.float32)
    for kb in range(_K // _KB):
        blk = cbt_bf[:, kb * _KB:(kb + 1) * _KB]
        ab = jnp.abs(blk[None, :, :] - xbc)
        t = ab[:, 0:_D // 2, :] + ab[:, _D // 2:, :]
        sabs = jnp.sum(t, axis=1, dtype=jnp.float32)
        accm = accm + sabs * z[:, kb * _KB:(kb + 1) * _KB]
    dpart_ref[...] = jnp.broadcast_to(jnp.sum(accm), (1, 1, 128))


def kernel(x, t, codebook, W1, b1, W2, b2):
    x = x.reshape(_B, -1)
    min_tau, max_tau = 1e-8, 10.0
    warm = min_tau + 0.5 * (max_tau - min_tau) * (1.0 + math.cos(5 / max_tau * math.pi))
    tau = jnp.where(max_tau > t, warm, min_tau).astype(jnp.float32).reshape(1, 1)
    g = jnp.asarray(_G)

    logits, bmu, dparts = pl.pallas_call(
        _vq_body,
        grid=(_NB,),
        in_specs=[
            pl.BlockSpec(memory_space=pltpu.SMEM),
            pl.BlockSpec((_TB, _D), lambda i: (i, 0)),
            pl.BlockSpec((_K, _D), lambda i: (0, 0)),
            pl.BlockSpec((_D, _K), lambda i: (0, 0)),
            pl.BlockSpec((_D, _H), lambda i: (0, 0)),
            pl.BlockSpec((1, _H), lambda i: (0, 0)),
            pl.BlockSpec((_H, _K), lambda i: (0, 0)),
            pl.BlockSpec((1, _K), lambda i: (0, 0)),
            pl.BlockSpec((_TB, _K), lambda i: (i, 0)),
        ],
        out_specs=(
            pl.BlockSpec((_TB, _K), lambda i: (i, 0)),
            pl.BlockSpec((1, _TB, 1), lambda i: (i, 0, 0)),
            pl.BlockSpec((1, 1, 128), lambda i: (i, 0, 0)),
        ),
        out_shape=(
            jax.ShapeDtypeStruct((_B, _K), jnp.float32),
            jax.ShapeDtypeStruct((_NB, _TB, 1), jnp.int32),
            jax.ShapeDtypeStruct((_NB, 1, 128), jnp.float32),
        ),
    )(tau, x, codebook, codebook.T, W1, b1.reshape(1, _H), W2,
      b2.reshape(1, _K), g)

    bmu_index = bmu.reshape(_B)
    delta = jnp.sum(dparts[:, 0, 0]) * jnp.float32(1.0 / (_B * _K * _D))
    return (logits, bmu_index, delta)


# speedup vs baseline: 1.0289x; 1.0289x over previous
import math

import numpy as np
import jax
import jax.numpy as jnp
from jax.experimental import pallas as pl
from jax.experimental.pallas import tpu as pltpu

_B, _D, _K, _H = 512, 256, 1024, 32
_TB = 128
_NB = _B // _TB
_KB = 128


def _make_gumbel():
    n = _B * _K
    ks0, ks1 = np.uint32(0), np.uint32(42)
    ks2 = ks0 ^ ks1 ^ np.uint32(0x1BD11BDA)
    x0 = np.full(n, ks0, np.uint32)
    x1 = np.arange(n, dtype=np.uint32)
    rot_a = np.uint32([13, 15, 26, 6])
    rot_b = np.uint32([17, 29, 16, 24])

    def rotl(v, d):
        return (v << d) | (v >> np.uint32(32 - d))

    with np.errstate(over="ignore"):
        x1 = x1 + ks1
        for rs, ka, kb, c in ((rot_a, ks1, ks2, 1), (rot_b, ks2, ks0, 2),
                              (rot_a, ks0, ks1, 3), (rot_b, ks1, ks2, 4),
                              (rot_a, ks2, ks0, 5)):
            for r in rs:
                x0 = x0 + x1
                x1 = x0 ^ rotl(x1, r)
            x0 = x0 + ka
            x1 = x1 + kb + np.uint32(c)
    bits = x0 ^ x1
    fb = (bits >> np.uint32(9)) | np.uint32(0x3F800000)
    tiny = np.float32(np.finfo(np.float32).tiny)
    u = np.maximum(tiny, (fb.view(np.float32) - np.float32(1.0))
                   * (np.float32(1.0) - tiny) + tiny)
    return (-np.log(-np.log(u))).astype(np.float32).reshape(_B, _K)


_G = _make_gumbel()


def _vq_body(tau_ref, x_ref, cb_ref, cbt_ref, w1_ref, b1_ref, w2_ref, b2_ref,
             g_ref, logits_ref, bmu_ref, dpart_ref):
    x = x_ref[...]

    h = jnp.maximum(
        jax.lax.dot_general(x, w1_ref[...], (((1,), (0,)), ((), ())),
                            preferred_element_type=jnp.float32)
        + b1_ref[...], 0.0)
    logits = jax.lax.dot_general(h, w2_ref[...], (((1,), (0,)), ((), ())),
                                 preferred_element_type=jnp.float32)
    logits = logits + b2_ref[...]
    logits_ref[...] = logits

    s = (logits + g_ref[...]) / tau_ref[0, 0]
    m = jnp.max(s, axis=1, keepdims=True)
    e = jnp.exp(s - m)
    z = e / jnp.sum(e, axis=1, keepdims=True)

    cbt = cbt_ref[...]
    cnorm = jnp.sum(cbt * cbt, axis=0, keepdims=True)
    dist = cnorm - 2.0 * jax.lax.dot_general(
        x, cbt, (((1,), (0,)), ((), ())),
        preferred_element_type=jnp.float32,
        precision=jax.lax.Precision.HIGHEST)
    iot = jax.lax.broadcasted_iota(jnp.int32, (_TB, _K), 1)
    m1 = jnp.min(dist, axis=1, keepdims=True)
    i1 = jnp.min(jnp.where(dist <= m1, iot, _K), axis=1, keepdims=True)
    dist2 = jnp.where(iot == i1, jnp.inf, dist)
    m2 = jnp.min(dist2, axis=1, keepdims=True)
    i2 = jnp.min(jnp.where(dist2 <= m2, iot, _K), axis=1, keepdims=True)
    ia = jnp.minimum(i1, i2)
    ib = jnp.maximum(i1, i2)
    cb = cb_ref[...]
    oha = (iot == ia).astype(jnp.float32)
    ohb = (iot == ib).astype(jnp.float32)
    ca = jax.lax.dot_general(oha, cb, (((1,), (0,)), ((), ())),
                             preferred_element_type=jnp.float32,
                             precision=jax.lax.Precision.HIGHEST)
    cbs = jax.lax.dot_general(ohb, cb, (((1,), (0,)), ((), ())),
                              preferred_element_type=jnp.float32,
                              precision=jax.lax.Precision.HIGHEST)
    da = ca - x
    db = cbs - x
    ssqa = jnp.sum(da * da, axis=1, keepdims=True)
    ssqb = jnp.sum(db * db, axis=1, keepdims=True)
    bmu_ref[...] = jnp.where(ssqa <= ssqb, ia, ib)[None]

    cbt_bf = cbt.astype(jnp.bfloat16)
    xbc = x.astype(jnp.bfloat16)[:, :, None]
    accm = jnp.zeros((_TB, _KB), jnp.float32)
    for kb in range(_K // _KB):
        blk = cbt_bf[:, kb * _KB:(kb + 1) * _KB]
        ab = jnp.abs(blk[None, :, :] - xbc)
        t = ab[:, 0:_D // 2, :] + ab[:, _D // 2:, :]
        sabs = jnp.sum(t, axis=1, dtype=jnp.float32)
        accm = accm + sabs * z[:, kb * _KB:(kb + 1) * _KB]
    dpart_ref[...] = jnp.broadcast_to(jnp.sum(accm), (1, 1, 128))


def kernel(x, t, codebook, W1, b1, W2, b2):
    x = x.reshape(_B, -1)
    min_tau, max_tau = 1e-8, 10.0
    warm = min_tau + 0.5 * (max_tau - min_tau) * (1.0 + math.cos(5 / max_tau * math.pi))
    tau = jnp.where(max_tau > t, warm, min_tau).astype(jnp.float32).reshape(1, 1)
    g = jnp.asarray(_G)

    logits, bmu, dparts = pl.pallas_call(
        _vq_body,
        grid=(_NB,),
        in_specs=[
            pl.BlockSpec(memory_space=pltpu.SMEM),
            pl.BlockSpec((_TB, _D), lambda i: (i, 0)),
            pl.BlockSpec((_K, _D), lambda i: (0, 0)),
            pl.BlockSpec((_D, _K), lambda i: (0, 0)),
            pl.BlockSpec((_D, _H), lambda i: (0, 0)),
            pl.BlockSpec((1, _H), lambda i: (0, 0)),
            pl.BlockSpec((_H, _K), lambda i: (0, 0)),
            pl.BlockSpec((1, _K), lambda i: (0, 0)),
            pl.BlockSpec((_TB, _K), lambda i: (i, 0)),
        ],
        out_specs=(
            pl.BlockSpec((_TB, _K), lambda i: (i, 0)),
            pl.BlockSpec((1, _TB, 1), lambda i: (i, 0, 0)),
            pl.BlockSpec((1, 1, 128), lambda i: (i, 0, 0)),
        ),
        out_shape=(
            jax.ShapeDtypeStruct((_B, _K), jnp.float32),
            jax.ShapeDtypeStruct((_NB, _TB, 1), jnp.int32),
            jax.ShapeDtypeStruct((_NB, 1, 128), jnp.float32),
        ),
    )(tau, x, codebook, codebook.T, W1, b1.reshape(1, _H), W2,
      b2.reshape(1, _K), g)

    bmu_index = bmu.reshape(_B)
    delta = jnp.sum(dparts[:, 0, 0]) * jnp.float32(1.0 / (_B * _K * _D))
    return (logits, bmu_index, delta)


# speedup vs baseline: 1.1889x; 1.1555x over previous
import math

import numpy as np
import jax
import jax.numpy as jnp
from jax.experimental import pallas as pl
from jax.experimental.pallas import tpu as pltpu

_B, _D, _K, _H = 512, 256, 1024, 32
_TB = 128
_NB = _B // _TB
_KB = 128


def _make_gumbel():
    n = _B * _K
    ks0, ks1 = np.uint32(0), np.uint32(42)
    ks2 = ks0 ^ ks1 ^ np.uint32(0x1BD11BDA)
    x0 = np.full(n, ks0, np.uint32)
    x1 = np.arange(n, dtype=np.uint32)
    rot_a = np.uint32([13, 15, 26, 6])
    rot_b = np.uint32([17, 29, 16, 24])

    def rotl(v, d):
        return (v << d) | (v >> np.uint32(32 - d))

    with np.errstate(over="ignore"):
        x1 = x1 + ks1
        for rs, ka, kb, c in ((rot_a, ks1, ks2, 1), (rot_b, ks2, ks0, 2),
                              (rot_a, ks0, ks1, 3), (rot_b, ks1, ks2, 4),
                              (rot_a, ks2, ks0, 5)):
            for r in rs:
                x0 = x0 + x1
                x1 = x0 ^ rotl(x1, r)
            x0 = x0 + ka
            x1 = x1 + kb + np.uint32(c)
    bits = x0 ^ x1
    fb = (bits >> np.uint32(9)) | np.uint32(0x3F800000)
    tiny = np.float32(np.finfo(np.float32).tiny)
    u = np.maximum(tiny, (fb.view(np.float32) - np.float32(1.0))
                   * (np.float32(1.0) - tiny) + tiny)
    return (-np.log(-np.log(u))).astype(np.float32).reshape(_B, _K)


_G = _make_gumbel()


def _vq_body(tau_ref, x_ref, cb_ref, cbt_ref, w1_ref, b1_ref, w2_ref, b2_ref,
             g_ref, logits_ref, bmu_ref, dpart_ref):
    x = x_ref[...]

    h = jnp.maximum(
        jax.lax.dot_general(x, w1_ref[...], (((1,), (0,)), ((), ())),
                            preferred_element_type=jnp.float32)
        + b1_ref[...], 0.0)
    logits = jax.lax.dot_general(h, w2_ref[...], (((1,), (0,)), ((), ())),
                                 preferred_element_type=jnp.float32)
    logits = logits + b2_ref[...]
    logits_ref[...] = logits

    s = (logits + g_ref[...]) / tau_ref[0, 0]
    m = jnp.max(s, axis=1, keepdims=True)
    e = jnp.exp(s - m)
    z = e / jnp.sum(e, axis=1, keepdims=True)

    cbt = cbt_ref[...]
    cnorm = jnp.sum(cbt * cbt, axis=0, keepdims=True)
    dist = cnorm - 2.0 * jax.lax.dot_general(
        x, cbt, (((1,), (0,)), ((), ())),
        preferred_element_type=jnp.float32,
        precision=jax.lax.Precision.HIGHEST)
    iot = jax.lax.broadcasted_iota(jnp.int32, (_TB, _K), 1)
    m1 = jnp.min(dist, axis=1, keepdims=True)
    i1 = jnp.min(jnp.where(dist <= m1, iot, _K), axis=1, keepdims=True)
    dist2 = jnp.where(iot == i1, jnp.inf, dist)
    m2 = jnp.min(dist2, axis=1, keepdims=True)
    i2 = jnp.min(jnp.where(dist2 <= m2, iot, _K), axis=1, keepdims=True)
    ia = jnp.minimum(i1, i2)
    ib = jnp.maximum(i1, i2)
    cb = cb_ref[...]
    oha = (iot == ia).astype(jnp.float32)
    ohb = (iot == ib).astype(jnp.float32)
    ca = jax.lax.dot_general(oha, cb, (((1,), (0,)), ((), ())),
                             preferred_element_type=jnp.float32,
                             precision=jax.lax.Precision.HIGHEST)
    cbs = jax.lax.dot_general(ohb, cb, (((1,), (0,)), ((), ())),
                              preferred_element_type=jnp.float32,
                              precision=jax.lax.Precision.HIGHEST)
    da = ca - x
    db = cbs - x
    ssqa = jnp.sum(da * da, axis=1, keepdims=True)
    ssqb = jnp.sum(db * db, axis=1, keepdims=True)
    bmu_ref[...] = jnp.where(ssqa <= ssqb, ia, ib)[None]

    cbt_bf = cbt.astype(jnp.bfloat16)
    xbc = x.astype(jnp.bfloat16)[:, :, None]
    accm = jnp.zeros((_TB, _KB), jnp.float32)
    for kb in range(_K // _KB):
        blk = cbt_bf[:, kb * _KB:(kb + 1) * _KB]
        ab = jnp.abs(blk[None, :, :] - xbc)
        t = ab[:, 0:128, :] + ab[:, 128:256, :]
        t = t[:, 0:64, :] + t[:, 64:128, :]
        t = t[:, 0:32, :] + t[:, 32:64, :]
        t = t[:, 0:16, :] + t[:, 16:32, :]
        sabs = jnp.sum(t, axis=1, dtype=jnp.float32)
        accm = accm + sabs * z[:, kb * _KB:(kb + 1) * _KB]
    dpart_ref[...] = jnp.broadcast_to(jnp.sum(accm), (1, 1, 128))


def kernel(x, t, codebook, W1, b1, W2, b2):
    x = x.reshape(_B, -1)
    min_tau, max_tau = 1e-8, 10.0
    warm = min_tau + 0.5 * (max_tau - min_tau) * (1.0 + math.cos(5 / max_tau * math.pi))
    tau = jnp.where(max_tau > t, warm, min_tau).astype(jnp.float32).reshape(1, 1)
    g = jnp.asarray(_G)

    logits, bmu, dparts = pl.pallas_call(
        _vq_body,
        grid=(_NB,),
        in_specs=[
            pl.BlockSpec(memory_space=pltpu.SMEM),
            pl.BlockSpec((_TB, _D), lambda i: (i, 0)),
            pl.BlockSpec((_K, _D), lambda i: (0, 0)),
            pl.BlockSpec((_D, _K), lambda i: (0, 0)),
            pl.BlockSpec((_D, _H), lambda i: (0, 0)),
            pl.BlockSpec((1, _H), lambda i: (0, 0)),
            pl.BlockSpec((_H, _K), lambda i: (0, 0)),
            pl.BlockSpec((1, _K), lambda i: (0, 0)),
            pl.BlockSpec((_TB, _K), lambda i: (i, 0)),
        ],
        out_specs=(
            pl.BlockSpec((_TB, _K), lambda i: (i, 0)),
            pl.BlockSpec((1, _TB, 1), lambda i: (i, 0, 0)),
            pl.BlockSpec((1, 1, 128), lambda i: (i, 0, 0)),
        ),
        out_shape=(
            jax.ShapeDtypeStruct((_B, _K), jnp.float32),
            jax.ShapeDtypeStruct((_NB, _TB, 1), jnp.int32),
            jax.ShapeDtypeStruct((_NB, 1, 128), jnp.float32),
        ),
    )(tau, x, codebook, codebook.T, W1, b1.reshape(1, _H), W2,
      b2.reshape(1, _K), g)

    bmu_index = bmu.reshape(_B)
    delta = jnp.sum(dparts[:, 0, 0]) * jnp.float32(1.0 / (_B * _K * _D))
    return (logits, bmu_index, delta)


# speedup vs baseline: 1.2654x; 1.0643x over previous
import math

import numpy as np
import jax
import jax.numpy as jnp
from jax.experimental import pallas as pl
from jax.experimental.pallas import tpu as pltpu

_B, _D, _K, _H = 512, 256, 1024, 32
_TB = 256
_NB = _B // _TB
_KB = 128


def _make_gumbel():
    n = _B * _K
    ks0, ks1 = np.uint32(0), np.uint32(42)
    ks2 = ks0 ^ ks1 ^ np.uint32(0x1BD11BDA)
    x0 = np.full(n, ks0, np.uint32)
    x1 = np.arange(n, dtype=np.uint32)
    rot_a = np.uint32([13, 15, 26, 6])
    rot_b = np.uint32([17, 29, 16, 24])

    def rotl(v, d):
        return (v << d) | (v >> np.uint32(32 - d))

    with np.errstate(over="ignore"):
        x1 = x1 + ks1
        for rs, ka, kb, c in ((rot_a, ks1, ks2, 1), (rot_b, ks2, ks0, 2),
                              (rot_a, ks0, ks1, 3), (rot_b, ks1, ks2, 4),
                              (rot_a, ks2, ks0, 5)):
            for r in rs:
                x0 = x0 + x1
                x1 = x0 ^ rotl(x1, r)
            x0 = x0 + ka
            x1 = x1 + kb + np.uint32(c)
    bits = x0 ^ x1
    fb = (bits >> np.uint32(9)) | np.uint32(0x3F800000)
    tiny = np.float32(np.finfo(np.float32).tiny)
    u = np.maximum(tiny, (fb.view(np.float32) - np.float32(1.0))
                   * (np.float32(1.0) - tiny) + tiny)
    return (-np.log(-np.log(u))).astype(np.float32).reshape(_B, _K)


_G = _make_gumbel()


def _vq_body(tau_ref, x_ref, cb_ref, cbt_ref, w1_ref, b1_ref, w2_ref, b2_ref,
             g_ref, logits_ref, bmu_ref, dpart_ref):
    x = x_ref[...]

    h = jnp.maximum(
        jax.lax.dot_general(x, w1_ref[...], (((1,), (0,)), ((), ())),
                            preferred_element_type=jnp.float32)
        + b1_ref[...], 0.0)
    logits = jax.lax.dot_general(h, w2_ref[...], (((1,), (0,)), ((), ())),
                                 preferred_element_type=jnp.float32)
    logits = logits + b2_ref[...]
    logits_ref[...] = logits

    s = (logits + g_ref[...]) / tau_ref[0, 0]
    m = jnp.max(s, axis=1, keepdims=True)
    e = jnp.exp(s - m)
    z = e / jnp.sum(e, axis=1, keepdims=True)

    cbt = cbt_ref[...]
    cnorm = jnp.sum(cbt * cbt, axis=0, keepdims=True)
    dist = cnorm - 2.0 * jax.lax.dot_general(
        x, cbt, (((1,), (0,)), ((), ())),
        preferred_element_type=jnp.float32,
        precision=jax.lax.Precision.HIGHEST)
    iot = jax.lax.broadcasted_iota(jnp.int32, (_TB, _K), 1)
    m1 = jnp.min(dist, axis=1, keepdims=True)
    i1 = jnp.min(jnp.where(dist <= m1, iot, _K), axis=1, keepdims=True)
    dist2 = jnp.where(iot == i1, jnp.inf, dist)
    m2 = jnp.min(dist2, axis=1, keepdims=True)
    i2 = jnp.min(jnp.where(dist2 <= m2, iot, _K), axis=1, keepdims=True)
    ia = jnp.minimum(i1, i2)
    ib = jnp.maximum(i1, i2)
    cb = cb_ref[...]
    oha = (iot == ia).astype(jnp.float32)
    ohb = (iot == ib).astype(jnp.float32)
    ca = jax.lax.dot_general(oha, cb, (((1,), (0,)), ((), ())),
                             preferred_element_type=jnp.float32,
                             precision=jax.lax.Precision.HIGHEST)
    cbs = jax.lax.dot_general(ohb, cb, (((1,), (0,)), ((), ())),
                              preferred_element_type=jnp.float32,
                              precision=jax.lax.Precision.HIGHEST)
    da = ca - x
    db = cbs - x
    ssqa = jnp.sum(da * da, axis=1, keepdims=True)
    ssqb = jnp.sum(db * db, axis=1, keepdims=True)
    bmu_ref[...] = jnp.where(ssqa <= ssqb, ia, ib)[None]

    cbt_bf = cbt.astype(jnp.bfloat16)
    xbc = x.astype(jnp.bfloat16)[:, :, None]
    accm = jnp.zeros((_TB, _KB), jnp.float32)
    for kb in range(_K // _KB):
        blk = cbt_bf[:, kb * _KB:(kb + 1) * _KB]
        ab = jnp.abs(blk[None, :, :] - xbc)
        t = ab[:, 0:128, :] + ab[:, 128:256, :]
        t = t[:, 0:64, :] + t[:, 64:128, :]
        t = t[:, 0:32, :] + t[:, 32:64, :]
        t = t[:, 0:16, :] + t[:, 16:32, :]
        sabs = jnp.sum(t, axis=1, dtype=jnp.float32)
        accm = accm + sabs * z[:, kb * _KB:(kb + 1) * _KB]
    dpart_ref[...] = jnp.broadcast_to(jnp.sum(accm), (1, 1, 128))


def kernel(x, t, codebook, W1, b1, W2, b2):
    x = x.reshape(_B, -1)
    min_tau, max_tau = 1e-8, 10.0
    warm = min_tau + 0.5 * (max_tau - min_tau) * (1.0 + math.cos(5 / max_tau * math.pi))
    tau = jnp.where(max_tau > t, warm, min_tau).astype(jnp.float32).reshape(1, 1)
    g = jnp.asarray(_G)

    logits, bmu, dparts = pl.pallas_call(
        _vq_body,
        grid=(_NB,),
        in_specs=[
            pl.BlockSpec(memory_space=pltpu.SMEM),
            pl.BlockSpec((_TB, _D), lambda i: (i, 0)),
            pl.BlockSpec((_K, _D), lambda i: (0, 0)),
            pl.BlockSpec((_D, _K), lambda i: (0, 0)),
            pl.BlockSpec((_D, _H), lambda i: (0, 0)),
            pl.BlockSpec((1, _H), lambda i: (0, 0)),
            pl.BlockSpec((_H, _K), lambda i: (0, 0)),
            pl.BlockSpec((1, _K), lambda i: (0, 0)),
            pl.BlockSpec((_TB, _K), lambda i: (i, 0)),
        ],
        out_specs=(
            pl.BlockSpec((_TB, _K), lambda i: (i, 0)),
            pl.BlockSpec((1, _TB, 1), lambda i: (i, 0, 0)),
            pl.BlockSpec((1, 1, 128), lambda i: (i, 0, 0)),
        ),
        out_shape=(
            jax.ShapeDtypeStruct((_B, _K), jnp.float32),
            jax.ShapeDtypeStruct((_NB, _TB, 1), jnp.int32),
            jax.ShapeDtypeStruct((_NB, 1, 128), jnp.float32),
        ),
    )(tau, x, codebook, codebook.T, W1, b1.reshape(1, _H), W2,
      b2.reshape(1, _K), g)

    bmu_index = bmu.reshape(_B)
    delta = jnp.sum(dparts[:, 0, 0]) * jnp.float32(1.0 / (_B * _K * _D))
    return (logits, bmu_index, delta)


# speedup vs baseline: 1.3166x; 1.0405x over previous
import math

import numpy as np
import jax
import jax.numpy as jnp
from jax.experimental import pallas as pl
from jax.experimental.pallas import tpu as pltpu

_B, _D, _K, _H = 512, 256, 1024, 32
_TB = 512
_NB = _B // _TB
_KB = 128


def _make_gumbel():
    n = _B * _K
    ks0, ks1 = np.uint32(0), np.uint32(42)
    ks2 = ks0 ^ ks1 ^ np.uint32(0x1BD11BDA)
    x0 = np.full(n, ks0, np.uint32)
    x1 = np.arange(n, dtype=np.uint32)
    rot_a = np.uint32([13, 15, 26, 6])
    rot_b = np.uint32([17, 29, 16, 24])

    def rotl(v, d):
        return (v << d) | (v >> np.uint32(32 - d))

    with np.errstate(over="ignore"):
        x1 = x1 + ks1
        for rs, ka, kb, c in ((rot_a, ks1, ks2, 1), (rot_b, ks2, ks0, 2),
                              (rot_a, ks0, ks1, 3), (rot_b, ks1, ks2, 4),
                              (rot_a, ks2, ks0, 5)):
            for r in rs:
                x0 = x0 + x1
                x1 = x0 ^ rotl(x1, r)
            x0 = x0 + ka
            x1 = x1 + kb + np.uint32(c)
    bits = x0 ^ x1
    fb = (bits >> np.uint32(9)) | np.uint32(0x3F800000)
    tiny = np.float32(np.finfo(np.float32).tiny)
    u = np.maximum(tiny, (fb.view(np.float32) - np.float32(1.0))
                   * (np.float32(1.0) - tiny) + tiny)
    return (-np.log(-np.log(u))).astype(np.float32).reshape(_B, _K)


_G = _make_gumbel()


def _vq_body(tau_ref, x_ref, cb_ref, cbt_ref, w1_ref, b1_ref, w2_ref, b2_ref,
             g_ref, logits_ref, bmu_ref, dpart_ref):
    x = x_ref[...]

    h = jnp.maximum(
        jax.lax.dot_general(x, w1_ref[...], (((1,), (0,)), ((), ())),
                            preferred_element_type=jnp.float32)
        + b1_ref[...], 0.0)
    logits = jax.lax.dot_general(h, w2_ref[...], (((1,), (0,)), ((), ())),
                                 preferred_element_type=jnp.float32)
    logits = logits + b2_ref[...]
    logits_ref[...] = logits

    s = (logits + g_ref[...]) / tau_ref[0, 0]
    m = jnp.max(s, axis=1, keepdims=True)
    e = jnp.exp(s - m)
    z = e / jnp.sum(e, axis=1, keepdims=True)

    cbt = cbt_ref[...]
    cnorm = jnp.sum(cbt * cbt, axis=0, keepdims=True)
    dist = cnorm - 2.0 * jax.lax.dot_general(
        x, cbt, (((1,), (0,)), ((), ())),
        preferred_element_type=jnp.float32,
        precision=jax.lax.Precision.HIGHEST)
    iot = jax.lax.broadcasted_iota(jnp.int32, (_TB, _K), 1)
    m1 = jnp.min(dist, axis=1, keepdims=True)
    i1 = jnp.min(jnp.where(dist <= m1, iot, _K), axis=1, keepdims=True)
    dist2 = jnp.where(iot == i1, jnp.inf, dist)
    m2 = jnp.min(dist2, axis=1, keepdims=True)
    i2 = jnp.min(jnp.where(dist2 <= m2, iot, _K), axis=1, keepdims=True)
    ia = jnp.minimum(i1, i2)
    ib = jnp.maximum(i1, i2)
    cb = cb_ref[...]
    oha = (iot == ia).astype(jnp.float32)
    ohb = (iot == ib).astype(jnp.float32)
    ca = jax.lax.dot_general(oha, cb, (((1,), (0,)), ((), ())),
                             preferred_element_type=jnp.float32,
                             precision=jax.lax.Precision.HIGHEST)
    cbs = jax.lax.dot_general(ohb, cb, (((1,), (0,)), ((), ())),
                              preferred_element_type=jnp.float32,
                              precision=jax.lax.Precision.HIGHEST)
    da = ca - x
    db = cbs - x
    ssqa = jnp.sum(da * da, axis=1, keepdims=True)
    ssqb = jnp.sum(db * db, axis=1, keepdims=True)
    bmu_ref[...] = jnp.where(ssqa <= ssqb, ia, ib)[None]

    cbt_bf = cbt.astype(jnp.bfloat16)
    xbc = x.astype(jnp.bfloat16)[:, :, None]
    accm = jnp.zeros((_TB, _KB), jnp.float32)
    for kb in range(_K // _KB):
        blk = cbt_bf[:, kb * _KB:(kb + 1) * _KB]
        ab = jnp.abs(blk[None, :, :] - xbc)
        t = ab[:, 0:128, :] + ab[:, 128:256, :]
        t = t[:, 0:64, :] + t[:, 64:128, :]
        t = t[:, 0:32, :] + t[:, 32:64, :]
        t = t[:, 0:16, :] + t[:, 16:32, :]
        sabs = jnp.sum(t, axis=1, dtype=jnp.float32)
        accm = accm + sabs * z[:, kb * _KB:(kb + 1) * _KB]
    dpart_ref[...] = jnp.broadcast_to(jnp.sum(accm), (1, 1, 128))


def kernel(x, t, codebook, W1, b1, W2, b2):
    x = x.reshape(_B, -1)
    min_tau, max_tau = 1e-8, 10.0
    warm = min_tau + 0.5 * (max_tau - min_tau) * (1.0 + math.cos(5 / max_tau * math.pi))
    tau = jnp.where(max_tau > t, warm, min_tau).astype(jnp.float32).reshape(1, 1)
    g = jnp.asarray(_G)

    logits, bmu, dparts = pl.pallas_call(
        _vq_body,
        grid=(_NB,),
        in_specs=[
            pl.BlockSpec(memory_space=pltpu.SMEM),
            pl.BlockSpec((_TB, _D), lambda i: (i, 0)),
            pl.BlockSpec((_K, _D), lambda i: (0, 0)),
            pl.BlockSpec((_D, _K), lambda i: (0, 0)),
            pl.BlockSpec((_D, _H), lambda i: (0, 0)),
            pl.BlockSpec((1, _H), lambda i: (0, 0)),
            pl.BlockSpec((_H, _K), lambda i: (0, 0)),
            pl.BlockSpec((1, _K), lambda i: (0, 0)),
            pl.BlockSpec((_TB, _K), lambda i: (i, 0)),
        ],
        out_specs=(
            pl.BlockSpec((_TB, _K), lambda i: (i, 0)),
            pl.BlockSpec((1, _TB, 1), lambda i: (i, 0, 0)),
            pl.BlockSpec((1, 1, 128), lambda i: (i, 0, 0)),
        ),
        out_shape=(
            jax.ShapeDtypeStruct((_B, _K), jnp.float32),
            jax.ShapeDtypeStruct((_NB, _TB, 1), jnp.int32),
            jax.ShapeDtypeStruct((_NB, 1, 128), jnp.float32),
        ),
    )(tau, x, codebook, codebook.T, W1, b1.reshape(1, _H), W2,
      b2.reshape(1, _K), g)

    bmu_index = bmu.reshape(_B)
    delta = jnp.sum(dparts[:, 0, 0]) * jnp.float32(1.0 / (_B * _K * _D))
    return (logits, bmu_index, delta)
